# baseline (speedup 1.0000x reference)
import functools

import numpy as np
import jax
import jax.numpy as jnp
from jax import lax
from jax.experimental import pallas as pl
from jax.experimental.pallas import tpu as pltpu
from jax.experimental.pallas import tpu_sc as plsc

_ROW_DEPTH = 30
_LANES = 16
_OUT_CHUNKS = 8
_NBUF_OUT = 4


def _packed_perm(n_seq: int, r_depth: int) -> np.ndarray:
    n = int(np.ceil(n_seq / r_depth) * r_depth)
    nb_rows = n // r_depth
    ind = np.arange(n, dtype=np.int32)
    perm = ind.reshape(nb_rows, r_depth).T.reshape(-1)
    perm = perm[perm < n_seq].astype(np.int32)
    p = perm.reshape(-1, 2, _LANES)
    return (p[:, 0, :] | (p[:, 1, :] << 16)).astype(np.int32).reshape(-1)


@functools.cache
def _build(batch: int, n_seq: int):
    info = plsc.get_sparse_core_info()
    n_workers = info.num_cores * info.num_subcores
    assert batch % n_workers == 0
    assert n_seq % (2 * _LANES * _OUT_CHUNKS) == 0
    rows_per_worker = batch // n_workers
    chunk = n_seq // _OUT_CHUNKS
    pairs_per_chunk = chunk // (2 * _LANES)

    mesh = plsc.VectorSubcoreMesh(core_axis_name="c", subcore_axis_name="s")

    @functools.partial(
        pl.kernel,
        mesh=mesh,
        out_type=jax.ShapeDtypeStruct((batch, n_seq), jnp.float32),
        scratch_types=[
            pltpu.VMEM((n_seq // 2,), jnp.int32),
            pltpu.VMEM((n_seq,), jnp.float32),
            pltpu.VMEM((n_seq,), jnp.float32),
            pltpu.VMEM((chunk,), jnp.float32),
            pltpu.VMEM((chunk,), jnp.float32),
            pltpu.VMEM((chunk,), jnp.float32),
            pltpu.VMEM((chunk,), jnp.float32),
            pltpu.SemaphoreType.DMA,
            pltpu.SemaphoreType.DMA,
            pltpu.SemaphoreType.DMA,
            pltpu.SemaphoreType.DMA,
            pltpu.SemaphoreType.DMA,
            pltpu.SemaphoreType.DMA,
        ],
        compiler_params=pltpu.CompilerParams(needs_layout_passes=False),
    )
    def interleave(in_hbm, pp_hbm, out_hbm, pp_v, in_v0, in_v1,
                   out_v0, out_v1, out_v2, out_v3,
                   sem_i0, sem_i1, sem_o0, sem_o1, sem_o2, sem_o3):
        wid = lax.axis_index("s") * info.num_cores + lax.axis_index("c")
        row0 = wid * rows_per_worker
        in_bufs, sem_ins = [in_v0, in_v1], [sem_i0, sem_i1]
        out_bufs = [out_v0, out_v1, out_v2, out_v3]
        sem_outs = [sem_o0, sem_o1, sem_o2, sem_o3]

        h_in = [None, None]
        h_out = [None] * _NBUF_OUT
        h_in[0] = pltpu.async_copy(in_hbm.at[row0], in_bufs[0], sem_ins[0])
        pltpu.sync_copy(pp_hbm, pp_v)

        for j in range(rows_per_worker):
            jb = j % 2
            h_in[jb].wait()
            if j + 1 < rows_per_worker:
                nb2 = (j + 1) % 2
                h_in[nb2] = pltpu.async_copy(
                    in_hbm.at[row0 + j + 1], in_bufs[nb2], sem_ins[nb2]
                )
            src = in_bufs[jb]
            for k in range(_OUT_CHUNKS):
                b = (j * _OUT_CHUNKS + k) % _NBUF_OUT
                if h_out[b] is not None:
                    h_out[b].wait()
                dst = out_bufs[b]
                pbase = k * chunk // 2

                @plsc.parallel_loop(0, pairs_per_chunk, unroll=8)
                def gather32(q, _dst=dst, _src=src, _pb=pbase):
                    v = pp_v[pl.ds(_pb + q * _LANES, _LANES)]
                    lo = v & jnp.int32(0xFFFF)
                    hi = lax.shift_right_logical(v, jnp.int32(16))
                    _dst[pl.ds(q * 2 * _LANES, _LANES)] = plsc.load_gather(
                        _src, [lo]
                    )
                    _dst[pl.ds(q * 2 * _LANES + _LANES, _LANES)] = (
                        plsc.load_gather(_src, [hi])
                    )

                h_out[b] = pltpu.async_copy(
                    dst,
                    out_hbm.at[row0 + j, pl.ds(k * chunk, chunk)],
                    sem_outs[b],
                )
        for b in range(_NBUF_OUT):
            h_out[b].wait()

    return interleave


def kernel(inputs):
    batch, n_seq = inputs.shape
    packed = jnp.asarray(_packed_perm(n_seq, _ROW_DEPTH))
    return _build(batch, n_seq)(inputs, packed)


# speedup vs baseline: 1.0951x; 1.0951x over previous
import functools

import numpy as np
import jax
import jax.numpy as jnp
from jax import lax
from jax.experimental import pallas as pl
from jax.experimental.pallas import tpu as pltpu
from jax.experimental.pallas import tpu_sc as plsc

_ROW_DEPTH = 30
_LANES = 16
_OUT_CHUNKS = 2
_NBUF_OUT = 2


def _packed_perm(n_seq: int, r_depth: int) -> np.ndarray:
    n = int(np.ceil(n_seq / r_depth) * r_depth)
    nb_rows = n // r_depth
    ind = np.arange(n, dtype=np.int32)
    perm = ind.reshape(nb_rows, r_depth).T.reshape(-1)
    perm = perm[perm < n_seq].astype(np.int32)
    p = perm.reshape(-1, 2, _LANES)
    return (p[:, 0, :] | (p[:, 1, :] << 16)).astype(np.int32).reshape(-1)


@functools.cache
def _build(batch: int, n_seq: int):
    info = plsc.get_sparse_core_info()
    n_workers = info.num_cores * info.num_subcores
    assert batch % n_workers == 0
    assert n_seq % (2 * _LANES * _OUT_CHUNKS) == 0
    rows_per_worker = batch // n_workers
    chunk = n_seq // _OUT_CHUNKS
    pairs_per_chunk = chunk // (2 * _LANES)

    mesh = plsc.VectorSubcoreMesh(core_axis_name="c", subcore_axis_name="s")

    @functools.partial(
        pl.kernel,
        mesh=mesh,
        out_type=jax.ShapeDtypeStruct((batch, n_seq), jnp.float32),
        scratch_types=[
            pltpu.VMEM((n_seq // 2,), jnp.int32),
            pltpu.VMEM((n_seq,), jnp.float32),
            pltpu.VMEM((n_seq,), jnp.float32),
            pltpu.VMEM((chunk,), jnp.float32),
            pltpu.VMEM((chunk,), jnp.float32),
            pltpu.SemaphoreType.DMA,
            pltpu.SemaphoreType.DMA,
            pltpu.SemaphoreType.DMA,
            pltpu.SemaphoreType.DMA,
        ],
        compiler_params=pltpu.CompilerParams(needs_layout_passes=False),
    )
    def interleave(in_hbm, pp_hbm, out_hbm, pp_v, in_v0, in_v1,
                   out_v0, out_v1,
                   sem_i0, sem_i1, sem_o0, sem_o1):
        wid = lax.axis_index("s") * info.num_cores + lax.axis_index("c")
        row0 = wid * rows_per_worker
        in_bufs, sem_ins = [in_v0, in_v1], [sem_i0, sem_i1]
        out_bufs = [out_v0, out_v1]
        sem_outs = [sem_o0, sem_o1]

        h_in = [None, None]
        h_out = [None] * _NBUF_OUT
        h_in[0] = pltpu.async_copy(in_hbm.at[row0], in_bufs[0], sem_ins[0])
        pltpu.sync_copy(pp_hbm, pp_v)

        for j in range(rows_per_worker):
            jb = j % 2
            h_in[jb].wait()
            if j + 1 < rows_per_worker:
                nb2 = (j + 1) % 2
                h_in[nb2] = pltpu.async_copy(
                    in_hbm.at[row0 + j + 1], in_bufs[nb2], sem_ins[nb2]
                )
            src = in_bufs[jb]
            for k in range(_OUT_CHUNKS):
                b = (j * _OUT_CHUNKS + k) % _NBUF_OUT
                if h_out[b] is not None:
                    h_out[b].wait()
                dst = out_bufs[b]
                pbase = k * chunk // 2

                @plsc.parallel_loop(0, pairs_per_chunk, unroll=8)
                def gather32(q, _dst=dst, _src=src, _pb=pbase):
                    v = pp_v[pl.ds(_pb + q * _LANES, _LANES)]
                    lo = v & jnp.int32(0xFFFF)
                    hi = lax.shift_right_logical(v, jnp.int32(16))
                    _dst[pl.ds(q * 2 * _LANES, _LANES)] = plsc.load_gather(
                        _src, [lo]
                    )
                    _dst[pl.ds(q * 2 * _LANES + _LANES, _LANES)] = (
                        plsc.load_gather(_src, [hi])
                    )

                h_out[b] = pltpu.async_copy(
                    dst,
                    out_hbm.at[row0 + j, pl.ds(k * chunk, chunk)],
                    sem_outs[b],
                )
        for b in range(_NBUF_OUT):
            h_out[b].wait()

    return interleave


def kernel(inputs):
    batch, n_seq = inputs.shape
    packed = jnp.asarray(_packed_perm(n_seq, _ROW_DEPTH))
    return _build(batch, n_seq)(inputs, packed)
